# Initial kernel scaffold; baseline (speedup 1.0000x reference)
import jax
import jax.numpy as jnp
from jax.experimental import pallas as pl


def kernel(batch_inputs, batch_graph, W, b):
    raise NotImplementedError("write your pallas kernel here")


# speedup vs baseline: 6003.0452x; 6003.0452x over previous
import jax
import jax.numpy as jnp
from jax.experimental import pallas as pl


def _gcnn_kernel(adj_ref, x_ref, w_ref, b_ref, out_ref):
    a = adj_ref[...].astype(jnp.float32)
    deg = jnp.sum(a, axis=0, keepdims=True) + 1.0
    d = jax.lax.rsqrt(deg)
    xw = jnp.dot(x_ref[...], w_ref[...], preferred_element_type=jnp.float32)
    y = xw * d.reshape(-1, 1)
    z = jax.lax.dot_general(a, y, (((0,), (0,)), ((), ())),
                            preferred_element_type=jnp.float32)
    out_ref[...] = (z + y) * d.reshape(-1, 1) + b_ref[...]


def kernel(batch_inputs, batch_graph, W, b):
    n, f = batch_inputs.shape
    return pl.pallas_call(
        _gcnn_kernel,
        out_shape=jax.ShapeDtypeStruct((n, W.shape[1]), batch_inputs.dtype),
    )(batch_graph, batch_inputs, W, b.reshape(1, -1))
